# baseline (speedup 1.0000x reference)
import functools

import jax
import jax.numpy as jnp
from jax import lax
from jax.experimental import pallas as pl
from jax.experimental.pallas import tpu as pltpu
from jax.experimental.pallas import tpu_sc as plsc

B, S, T = 16, 512, 64
G = T // 16
LN2 = 0.6931471805599453
SQRT2 = 1.4142135623730951


def _group(ref, row, g):
    return ref[row, pl.ds(g * 16, 16)]


def _vmax4(vs):
    return jnp.maximum(jnp.maximum(vs[0], vs[1]), jnp.maximum(vs[2], vs[3]))


def _crf_body(em_h, tok_h, tr_h, hd_h, la_h, out_h,
              em_v, tr_v, trb_v, tok_v, hd_v, la_v, row_v, sem):
    c = lax.axis_index("c")
    s = lax.axis_index("s")
    b = s

    em_cp = pltpu.async_copy(em_h.at[b], em_v, sem)
    pltpu.sync_copy(tok_h, tok_v)
    pltpu.sync_copy(tr_h, tr_v)
    pltpu.sync_copy(hd_h, hd_v)
    pltpu.sync_copy(la_h, la_v)

    tok_vec = tok_v[pl.ds(0, 16)]
    seq_len = tok_vec[jnp.full((16,), b, jnp.int32)][0]

    def mT_step(i, mv):
        for g in range(G):
            mv = jnp.maximum(mv, _group(tr_v, i, g))
        return mv

    mT_vec = lax.fori_loop(0, T, mT_step, jnp.full((16,), -jnp.inf, jnp.float32))
    mT = jnp.max(mT_vec)

    def exp_step(i, carry):
        eg = [jnp.exp(_group(tr_v, i, g) - mT) for g in range(G)]
        b01 = plsc.pack(eg[0], eg[1], format=plsc.PackFormat.INTERLEAVED)
        b23 = plsc.pack(eg[2], eg[3], format=plsc.PackFormat.INTERLEAVED)
        trb_v[i, pl.ds(0, 16)] = plsc.bitcast(b01, jnp.int32)
        trb_v[i, pl.ds(16, 16)] = plsc.bitcast(b23, jnp.int32)
        return carry

    lax.fori_loop(0, T, exp_step, 0)
    em_cp.wait()

    c0 = [hd_v[pl.ds(g * 16, 16)] + _group(em_v, 0, g) for g in range(G)]
    m0 = jnp.max(_vmax4(c0))
    p_init = tuple(jnp.exp(c0[g] - m0) for g in range(G))

    NPRE = 16
    er = [(plsc.bitcast(trb_v[i, pl.ds(0, 16)], jnp.bfloat16),
           plsc.bitcast(trb_v[i, pl.ds(16, 16)], jnp.bfloat16))
          for i in range(NPRE)]

    def step(t, carry):
        k_acc, kv, p0, p1, p2, p3 = carry
        pc = (p0, p1, p2, p3)
        sp = [plsc.bitcast(
            plsc.pack(pc[g], pc[g], format=plsc.PackFormat.INTERLEAVED),
            jnp.int32) for g in range(G)]
        scale = plsc.bitcast((127 - kv) << 23, jnp.float32)
        e = [_group(em_v, t, g) for g in range(G)]
        ws = [jnp.exp(jnp.minimum(e[g], 80.0)) * scale for g in range(G)]
        q01 = jnp.zeros((32,), jnp.bfloat16)
        q23 = jnp.zeros((32,), jnp.bfloat16)
        for lane in range(16):
            idx = jnp.full((16,), lane, jnp.int32)
            for gs in range(G):
                pib = plsc.bitcast(sp[gs][idx], jnp.bfloat16)
                i = gs * 16 + lane
                if i < NPRE:
                    e01, e23 = er[i]
                else:
                    e01 = plsc.bitcast(trb_v[i, pl.ds(0, 16)], jnp.bfloat16)
                    e23 = plsc.bitcast(trb_v[i, pl.ds(16, 16)], jnp.bfloat16)
                q01 = q01 + pib * e01
                q23 = q23 + pib * e23
        q0, q1 = plsc.unpack(q01, format=plsc.PackFormat.INTERLEAVED)
        q2, q3 = plsc.unpack(q23, format=plsc.PackFormat.INTERLEAVED)
        q = [q0, q1, q2, q3]
        pnew = [q[g] * ws[g] for g in range(G)]
        pm = jnp.max(_vmax4(pnew))
        bits = plsc.bitcast(jnp.full((16,), pm), jnp.int32)
        kv_new = (bits >> 23) - 127
        return (k_acc + kv, kv_new, pnew[0], pnew[1], pnew[2], pnew[3])

    k_acc, _, f0, f1, f2, f3 = lax.fori_loop(
        1, seq_len, step,
        (jnp.zeros((16,), jnp.int32), jnp.zeros((16,), jnp.int32)) + p_init)
    pf = (f0, f1, f2, f3)
    s_acc = m0 + (seq_len - 1).astype(jnp.float32) * mT

    lg = [la_v[pl.ds(g * 16, 16)] for g in range(G)]
    mL = jnp.max(_vmax4(lg))
    z = [pf[g] * jnp.exp(lg[g] - mL) for g in range(G)]
    Z = jnp.sum(z[0] + z[1] + z[2] + z[3])

    zbits = plsc.bitcast(jnp.full((16,), Z), jnp.int32)
    ev = (zbits >> 23) - 127
    mant = plsc.bitcast((zbits & 0x007FFFFF) | 0x3F800000, jnp.float32)
    big = mant > SQRT2
    mant = jnp.where(big, mant * 0.5, mant)
    ev = jnp.where(big, ev + 1, ev)
    tt = (mant - 1.0) / (mant + 1.0)
    t2 = tt * tt
    lnm = tt * (2.0 + t2 * (2.0 / 3.0 + t2 * (2.0 / 5.0
                + t2 * (2.0 / 7.0 + t2 * (2.0 / 9.0)))))
    res = lnm + (ev + k_acc).astype(jnp.float32) * LN2 + (s_acc + mL)
    row_v[...] = res

    @pl.when(c == 0)
    def _():
        pltpu.sync_copy(row_v, out_h.at[b])


def kernel(emissions, token_sizes, transitions, head_transitions,
           last_transitions):
    tok32 = token_sizes.astype(jnp.int32)
    mesh = plsc.VectorSubcoreMesh(core_axis_name="c", subcore_axis_name="s")
    run = functools.partial(
        pl.kernel,
        out_type=jax.ShapeDtypeStruct((B, 16), jnp.float32),
        mesh=mesh,
        scratch_types=[
            pltpu.VMEM((S, T), jnp.float32),
            pltpu.VMEM((T, T), jnp.float32),
            pltpu.VMEM((T, 32), jnp.int32),
            pltpu.VMEM((B,), jnp.int32),
            pltpu.VMEM((T,), jnp.float32),
            pltpu.VMEM((T,), jnp.float32),
            pltpu.VMEM((16,), jnp.float32),
            pltpu.SemaphoreType.DMA,
        ],
        compiler_params=pltpu.CompilerParams(needs_layout_passes=False),
    )(_crf_body)
    out = run(emissions, tok32, transitions, head_transitions,
              last_transitions)
    return out[:, 0]


# speedup vs baseline: 1.0535x; 1.0535x over previous
import functools

import jax
import jax.numpy as jnp
from jax import lax
from jax.experimental import pallas as pl
from jax.experimental.pallas import tpu as pltpu
from jax.experimental.pallas import tpu_sc as plsc

B, S, T = 16, 512, 64
G = T // 16
LN2 = 0.6931471805599453
SQRT2 = 1.4142135623730951


def _group(ref, row, g):
    return ref[row, pl.ds(g * 16, 16)]


def _vmax4(vs):
    return jnp.maximum(jnp.maximum(vs[0], vs[1]), jnp.maximum(vs[2], vs[3]))


def _crf_body(em_h, tok_h, tr_h, hd_h, la_h, out_h,
              em_v, tr_v, trb_v, tok_v, hd_v, la_v, row_v, sem):
    c = lax.axis_index("c")
    s = lax.axis_index("s")
    b = s

    em_cp = pltpu.async_copy(em_h.at[b], em_v, sem)
    pltpu.sync_copy(tok_h, tok_v)
    pltpu.sync_copy(tr_h, tr_v)
    pltpu.sync_copy(hd_h, hd_v)
    pltpu.sync_copy(la_h, la_v)

    tok_vec = tok_v[pl.ds(0, 16)]
    seq_len = tok_vec[jnp.full((16,), b, jnp.int32)][0]

    def mT_step(i, mv):
        for g in range(G):
            mv = jnp.maximum(mv, _group(tr_v, i, g))
        return mv

    mT_vec = lax.fori_loop(0, T, mT_step, jnp.full((16,), -jnp.inf, jnp.float32))
    mT = jnp.max(mT_vec)

    def exp_step(i, carry):
        eg = [jnp.exp(_group(tr_v, i, g) - mT) for g in range(G)]
        b01 = plsc.pack(eg[0], eg[1], format=plsc.PackFormat.INTERLEAVED)
        b23 = plsc.pack(eg[2], eg[3], format=plsc.PackFormat.INTERLEAVED)
        trb_v[i, pl.ds(0, 16)] = plsc.bitcast(b01, jnp.int32)
        trb_v[i, pl.ds(16, 16)] = plsc.bitcast(b23, jnp.int32)
        return carry

    lax.fori_loop(0, T, exp_step, 0)
    em_cp.wait()

    c0 = [hd_v[pl.ds(g * 16, 16)] + _group(em_v, 0, g) for g in range(G)]
    m0 = jnp.max(_vmax4(c0))
    p_init = tuple(jnp.exp(c0[g] - m0) for g in range(G))

    NPRE = 8
    er = [(plsc.bitcast(trb_v[i, pl.ds(0, 16)], jnp.bfloat16),
           plsc.bitcast(trb_v[i, pl.ds(16, 16)], jnp.bfloat16))
          for i in range(NPRE)]

    def step(t, carry):
        k_acc, kv, p0, p1, p2, p3 = carry
        pc = (p0, p1, p2, p3)
        sp = [plsc.bitcast(
            plsc.pack(pc[g], pc[g], format=plsc.PackFormat.INTERLEAVED),
            jnp.int32) for g in range(G)]
        scale = plsc.bitcast((127 - kv) << 23, jnp.float32)
        e = [_group(em_v, t, g) for g in range(G)]
        ws = [jnp.exp(jnp.minimum(e[g], 80.0)) * scale for g in range(G)]
        q01 = jnp.zeros((32,), jnp.bfloat16)
        q23 = jnp.zeros((32,), jnp.bfloat16)
        for lane in range(16):
            idx = jnp.full((16,), lane, jnp.int32)
            for gs in range(G):
                pib = plsc.bitcast(sp[gs][idx], jnp.bfloat16)
                i = gs * 16 + lane
                if i < NPRE:
                    e01, e23 = er[i]
                else:
                    e01 = plsc.bitcast(trb_v[i, pl.ds(0, 16)], jnp.bfloat16)
                    e23 = plsc.bitcast(trb_v[i, pl.ds(16, 16)], jnp.bfloat16)
                q01 = q01 + pib * e01
                q23 = q23 + pib * e23
        q0, q1 = plsc.unpack(q01, format=plsc.PackFormat.INTERLEAVED)
        q2, q3 = plsc.unpack(q23, format=plsc.PackFormat.INTERLEAVED)
        q = [q0, q1, q2, q3]
        pnew = [q[g] * ws[g] for g in range(G)]
        pm = jnp.max(_vmax4(pnew))
        bits = plsc.bitcast(jnp.full((16,), pm), jnp.int32)
        kv_new = (bits >> 23) - 127
        return (k_acc + kv, kv_new, pnew[0], pnew[1], pnew[2], pnew[3])

    k_acc, _, f0, f1, f2, f3 = lax.fori_loop(
        1, seq_len, step,
        (jnp.zeros((16,), jnp.int32), jnp.zeros((16,), jnp.int32)) + p_init)
    pf = (f0, f1, f2, f3)
    s_acc = m0 + (seq_len - 1).astype(jnp.float32) * mT

    lg = [la_v[pl.ds(g * 16, 16)] for g in range(G)]
    mL = jnp.max(_vmax4(lg))
    z = [pf[g] * jnp.exp(lg[g] - mL) for g in range(G)]
    Z = jnp.sum(z[0] + z[1] + z[2] + z[3])

    zbits = plsc.bitcast(jnp.full((16,), Z), jnp.int32)
    ev = (zbits >> 23) - 127
    mant = plsc.bitcast((zbits & 0x007FFFFF) | 0x3F800000, jnp.float32)
    big = mant > SQRT2
    mant = jnp.where(big, mant * 0.5, mant)
    ev = jnp.where(big, ev + 1, ev)
    tt = (mant - 1.0) / (mant + 1.0)
    t2 = tt * tt
    lnm = tt * (2.0 + t2 * (2.0 / 3.0 + t2 * (2.0 / 5.0
                + t2 * (2.0 / 7.0 + t2 * (2.0 / 9.0)))))
    res = lnm + (ev + k_acc).astype(jnp.float32) * LN2 + (s_acc + mL)
    row_v[...] = res

    @pl.when(c == 0)
    def _():
        pltpu.sync_copy(row_v, out_h.at[b])


def kernel(emissions, token_sizes, transitions, head_transitions,
           last_transitions):
    tok32 = token_sizes.astype(jnp.int32)
    mesh = plsc.VectorSubcoreMesh(core_axis_name="c", subcore_axis_name="s")
    run = functools.partial(
        pl.kernel,
        out_type=jax.ShapeDtypeStruct((B, 16), jnp.float32),
        mesh=mesh,
        scratch_types=[
            pltpu.VMEM((S, T), jnp.float32),
            pltpu.VMEM((T, T), jnp.float32),
            pltpu.VMEM((T, 32), jnp.int32),
            pltpu.VMEM((B,), jnp.int32),
            pltpu.VMEM((T,), jnp.float32),
            pltpu.VMEM((T,), jnp.float32),
            pltpu.VMEM((16,), jnp.float32),
            pltpu.SemaphoreType.DMA,
        ],
        compiler_params=pltpu.CompilerParams(needs_layout_passes=False),
    )(_crf_body)
    out = run(emissions, tok32, transitions, head_transitions,
              last_transitions)
    return out[:, 0]


# speedup vs baseline: 1.0742x; 1.0196x over previous
import functools

import jax
import jax.numpy as jnp
from jax import lax
from jax.experimental import pallas as pl
from jax.experimental.pallas import tpu as pltpu
from jax.experimental.pallas import tpu_sc as plsc

B, S, T = 16, 512, 64
G = T // 16
LN2 = 0.6931471805599453
SQRT2 = 1.4142135623730951


def _group(ref, row, g):
    return ref[row, pl.ds(g * 16, 16)]


def _vmax4(vs):
    return jnp.maximum(jnp.maximum(vs[0], vs[1]), jnp.maximum(vs[2], vs[3]))


def _crf_body(em_h, tok_h, tr_h, hd_h, la_h, out_h,
              em_v, tr_v, trb_v, tok_v, hd_v, la_v, row_v, sem):
    c = lax.axis_index("c")
    s = lax.axis_index("s")
    b = s

    em_cp = pltpu.async_copy(em_h.at[b], em_v, sem)
    pltpu.sync_copy(tok_h, tok_v)
    pltpu.sync_copy(tr_h, tr_v)
    pltpu.sync_copy(hd_h, hd_v)
    pltpu.sync_copy(la_h, la_v)

    tok_vec = tok_v[pl.ds(0, 16)]
    seq_len = tok_vec[jnp.full((16,), b, jnp.int32)][0]

    def mT_step(i, mv):
        for g in range(G):
            mv = jnp.maximum(mv, _group(tr_v, i, g))
        return mv

    mT_vec = lax.fori_loop(0, T, mT_step, jnp.full((16,), -jnp.inf, jnp.float32))
    mT = jnp.max(mT_vec)

    def exp_step(i, carry):
        eg = [jnp.exp(_group(tr_v, i, g) - mT) for g in range(G)]
        b01 = plsc.pack(eg[0], eg[1], format=plsc.PackFormat.INTERLEAVED)
        b23 = plsc.pack(eg[2], eg[3], format=plsc.PackFormat.INTERLEAVED)
        trb_v[i, pl.ds(0, 16)] = plsc.bitcast(b01, jnp.int32)
        trb_v[i, pl.ds(16, 16)] = plsc.bitcast(b23, jnp.int32)
        return carry

    lax.fori_loop(0, T, exp_step, 0)
    em_cp.wait()

    c0 = [hd_v[pl.ds(g * 16, 16)] + _group(em_v, 0, g) for g in range(G)]
    m0 = jnp.max(_vmax4(c0))
    p_init = tuple(jnp.exp(c0[g] - m0) for g in range(G))

    def step(t, carry):
        k_acc, kv, p0, p1, p2, p3 = carry
        pc = (p0, p1, p2, p3)
        sp = [plsc.bitcast(
            plsc.pack(pc[g], pc[g], format=plsc.PackFormat.INTERLEAVED),
            jnp.int32) for g in range(G)]
        scale = plsc.bitcast((127 - kv) << 23, jnp.float32)
        e = [_group(em_v, t, g) for g in range(G)]
        ws = [jnp.exp(jnp.minimum(e[g], 80.0)) * scale for g in range(G)]
        q01 = jnp.zeros((32,), jnp.bfloat16)
        q23 = jnp.zeros((32,), jnp.bfloat16)
        for lane in range(16):
            idx = jnp.full((16,), lane, jnp.int32)
            for gs in range(G):
                pib = plsc.bitcast(sp[gs][idx], jnp.bfloat16)
                i = gs * 16 + lane
                e01 = plsc.bitcast(trb_v[i, pl.ds(0, 16)], jnp.bfloat16)
                e23 = plsc.bitcast(trb_v[i, pl.ds(16, 16)], jnp.bfloat16)
                q01 = q01 + pib * e01
                q23 = q23 + pib * e23
        q0, q1 = plsc.unpack(q01, format=plsc.PackFormat.INTERLEAVED)
        q2, q3 = plsc.unpack(q23, format=plsc.PackFormat.INTERLEAVED)
        q = [q0, q1, q2, q3]
        pnew = [q[g] * ws[g] for g in range(G)]
        pm = jnp.max(_vmax4(pnew))
        bits = plsc.bitcast(jnp.full((16,), pm), jnp.int32)
        kv_new = (bits >> 23) - 127
        return (k_acc + kv, kv_new, pnew[0], pnew[1], pnew[2], pnew[3])

    k_acc, _, f0, f1, f2, f3 = lax.fori_loop(
        1, seq_len, step,
        (jnp.zeros((16,), jnp.int32), jnp.zeros((16,), jnp.int32)) + p_init)
    pf = (f0, f1, f2, f3)
    s_acc = m0 + (seq_len - 1).astype(jnp.float32) * mT

    lg = [la_v[pl.ds(g * 16, 16)] for g in range(G)]
    mL = jnp.max(_vmax4(lg))
    z = [pf[g] * jnp.exp(lg[g] - mL) for g in range(G)]
    Z = jnp.sum(z[0] + z[1] + z[2] + z[3])

    zbits = plsc.bitcast(jnp.full((16,), Z), jnp.int32)
    ev = (zbits >> 23) - 127
    mant = plsc.bitcast((zbits & 0x007FFFFF) | 0x3F800000, jnp.float32)
    big = mant > SQRT2
    mant = jnp.where(big, mant * 0.5, mant)
    ev = jnp.where(big, ev + 1, ev)
    tt = (mant - 1.0) / (mant + 1.0)
    t2 = tt * tt
    lnm = tt * (2.0 + t2 * (2.0 / 3.0 + t2 * (2.0 / 5.0
                + t2 * (2.0 / 7.0 + t2 * (2.0 / 9.0)))))
    res = lnm + (ev + k_acc).astype(jnp.float32) * LN2 + (s_acc + mL)
    row_v[...] = res

    @pl.when(c == 0)
    def _():
        pltpu.sync_copy(row_v, out_h.at[b])


def kernel(emissions, token_sizes, transitions, head_transitions,
           last_transitions):
    tok32 = token_sizes.astype(jnp.int32)
    mesh = plsc.VectorSubcoreMesh(core_axis_name="c", subcore_axis_name="s")
    run = functools.partial(
        pl.kernel,
        out_type=jax.ShapeDtypeStruct((B, 16), jnp.float32),
        mesh=mesh,
        scratch_types=[
            pltpu.VMEM((S, T), jnp.float32),
            pltpu.VMEM((T, T), jnp.float32),
            pltpu.VMEM((T, 32), jnp.int32),
            pltpu.VMEM((B,), jnp.int32),
            pltpu.VMEM((T,), jnp.float32),
            pltpu.VMEM((T,), jnp.float32),
            pltpu.VMEM((16,), jnp.float32),
            pltpu.SemaphoreType.DMA,
        ],
        compiler_params=pltpu.CompilerParams(needs_layout_passes=False),
    )(_crf_body)
    out = run(emissions, tok32, transitions, head_transitions,
              last_transitions)
    return out[:, 0]


# speedup vs baseline: 1.1340x; 1.0556x over previous
import functools

import jax
import jax.numpy as jnp
from jax import lax
from jax.experimental import pallas as pl
from jax.experimental.pallas import tpu as pltpu
from jax.experimental.pallas import tpu_sc as plsc

B, S, T = 16, 512, 64
G = T // 16
LN2 = 0.6931471805599453
SQRT2 = 1.4142135623730951


def _group(ref, row, g):
    return ref[row, pl.ds(g * 16, 16)]


def _vmax4(vs):
    return jnp.maximum(jnp.maximum(vs[0], vs[1]), jnp.maximum(vs[2], vs[3]))


def _crf_body(em_h, tok_h, tr_h, hd_h, la_h, out_h,
              em_v, tr_v, trb_v, tok_v, hd_v, la_v, row_v, sem):
    c = lax.axis_index("c")
    s = lax.axis_index("s")
    b = s

    em_cp = pltpu.async_copy(em_h.at[b], em_v, sem)
    pltpu.sync_copy(tok_h, tok_v)
    pltpu.sync_copy(tr_h, tr_v)
    pltpu.sync_copy(hd_h, hd_v)
    pltpu.sync_copy(la_h, la_v)

    tok_vec = tok_v[pl.ds(0, 16)]
    seq_len = tok_vec[jnp.full((16,), b, jnp.int32)][0]

    def mT_step(i, mv):
        for g in range(G):
            mv = jnp.maximum(mv, _group(tr_v, i, g))
        return mv

    mT_vec = lax.fori_loop(0, T, mT_step, jnp.full((16,), -jnp.inf, jnp.float32))
    mT = jnp.max(mT_vec)

    def exp_step(i, carry):
        eg = [jnp.exp(_group(tr_v, i, g) - mT) for g in range(G)]
        b01 = plsc.pack(eg[0], eg[1], format=plsc.PackFormat.INTERLEAVED)
        b23 = plsc.pack(eg[2], eg[3], format=plsc.PackFormat.INTERLEAVED)
        trb_v[i, pl.ds(0, 16)] = plsc.bitcast(b01, jnp.int32)
        trb_v[i, pl.ds(16, 16)] = plsc.bitcast(b23, jnp.int32)
        return carry

    lax.fori_loop(0, T, exp_step, 0)
    em_cp.wait()

    c0 = [hd_v[pl.ds(g * 16, 16)] + _group(em_v, 0, g) for g in range(G)]
    m0 = jnp.max(_vmax4(c0))
    p_init = tuple(jnp.exp(c0[g] - m0) for g in range(G))

    def step(t, carry):
        k_acc, kv, p0, p1, p2, p3 = carry
        pc = (p0, p1, p2, p3)
        sp = [plsc.bitcast(
            plsc.pack(pc[g], pc[g], format=plsc.PackFormat.INTERLEAVED),
            jnp.int32) for g in range(G)]
        scale = plsc.bitcast((127 - kv) << 23, jnp.float32)
        e = [_group(em_v, t, g) for g in range(G)]
        ws = [jnp.exp(jnp.minimum(e[g], 80.0)) * scale for g in range(G)]
        q01 = jnp.zeros((32,), jnp.bfloat16)
        q23 = jnp.zeros((32,), jnp.bfloat16)
        for lane in range(16):
            idx = jnp.full((16,), lane, jnp.int32)
            for gs in range(G):
                pib = plsc.bitcast(sp[gs][idx], jnp.bfloat16)
                i = gs * 16 + lane
                e01 = plsc.bitcast(trb_v[i, pl.ds(0, 16)], jnp.bfloat16)
                e23 = plsc.bitcast(trb_v[i, pl.ds(16, 16)], jnp.bfloat16)
                q01 = q01 + pib * e01
                q23 = q23 + pib * e23
        q0, q1 = plsc.unpack(q01, format=plsc.PackFormat.INTERLEAVED)
        q2, q3 = plsc.unpack(q23, format=plsc.PackFormat.INTERLEAVED)
        q = [q0, q1, q2, q3]
        pnew = [q[g] * ws[g] for g in range(G)]
        pm = jnp.max(_vmax4(pnew))
        bits = plsc.bitcast(jnp.full((16,), pm), jnp.int32)
        kv_new = (bits >> 23) - 127
        return (k_acc + kv, kv_new, pnew[0], pnew[1], pnew[2], pnew[3])

    k_acc, _, f0, f1, f2, f3 = lax.fori_loop(
        1, seq_len, step,
        (jnp.zeros((16,), jnp.int32), jnp.zeros((16,), jnp.int32)) + p_init)
    pf = (f0, f1, f2, f3)
    s_acc = m0 + (seq_len - 1).astype(jnp.float32) * mT

    lg = [la_v[pl.ds(g * 16, 16)] for g in range(G)]
    mL = jnp.max(_vmax4(lg))
    z = [pf[g] * jnp.exp(lg[g] - mL) for g in range(G)]
    Z = jnp.sum(z[0] + z[1] + z[2] + z[3])

    zbits = plsc.bitcast(jnp.full((16,), Z), jnp.int32)
    ev = (zbits >> 23) - 127
    mant = plsc.bitcast((zbits & 0x007FFFFF) | 0x3F800000, jnp.float32)
    big = mant > SQRT2
    mant = jnp.where(big, mant * 0.5, mant)
    ev = jnp.where(big, ev + 1, ev)
    tt = (mant - 1.0) / (mant + 1.0)
    t2 = tt * tt
    lnm = tt * (2.0 + t2 * (2.0 / 3.0 + t2 * (2.0 / 5.0
                + t2 * (2.0 / 7.0 + t2 * (2.0 / 9.0)))))
    res = lnm + (ev + k_acc).astype(jnp.float32) * LN2 + (s_acc + mL)
    row_v[...] = res

    @pl.when(c == 0)
    def _():
        pltpu.sync_copy(row_v, out_h.at[b])


def kernel(emissions, token_sizes, transitions, head_transitions,
           last_transitions):
    tok32 = token_sizes.astype(jnp.int32)
    mesh = plsc.VectorSubcoreMesh(core_axis_name="c", subcore_axis_name="s", num_cores=1)
    run = functools.partial(
        pl.kernel,
        out_type=jax.ShapeDtypeStruct((B, 16), jnp.float32),
        mesh=mesh,
        scratch_types=[
            pltpu.VMEM((S, T), jnp.float32),
            pltpu.VMEM((T, T), jnp.float32),
            pltpu.VMEM((T, 32), jnp.int32),
            pltpu.VMEM((B,), jnp.int32),
            pltpu.VMEM((T,), jnp.float32),
            pltpu.VMEM((T,), jnp.float32),
            pltpu.VMEM((16,), jnp.float32),
            pltpu.SemaphoreType.DMA,
        ],
        compiler_params=pltpu.CompilerParams(needs_layout_passes=False),
    )(_crf_body)
    out = run(emissions, tok32, transitions, head_transitions,
              last_transitions)
    return out[:, 0]


# speedup vs baseline: 1.1507x; 1.0147x over previous
import functools

import jax
import jax.numpy as jnp
from jax import lax
from jax.experimental import pallas as pl
from jax.experimental.pallas import tpu as pltpu
from jax.experimental.pallas import tpu_sc as plsc

B, S, T = 16, 512, 64
G = T // 16
LN2 = 0.6931471805599453
SQRT2 = 1.4142135623730951


def _group(ref, row, g):
    return ref[row, pl.ds(g * 16, 16)]


def _vmax4(vs):
    return jnp.maximum(jnp.maximum(vs[0], vs[1]), jnp.maximum(vs[2], vs[3]))


def _crf_body(em_h, tok_h, tr_h, hd_h, la_h, out_h,
              em_v, tr_v, trb_v, tok_v, hd_v, la_v, row_v, shr_v, gat_v, sem):
    c = lax.axis_index("c")
    s = lax.axis_index("s")
    b = s

    em_cp = pltpu.async_copy(em_h.at[b], em_v, sem)
    pltpu.sync_copy(tok_h, tok_v)
    pltpu.sync_copy(tr_h, tr_v)
    pltpu.sync_copy(hd_h, hd_v)
    pltpu.sync_copy(la_h, la_v)

    tok_vec = tok_v[pl.ds(0, 16)]
    seq_len = tok_vec[jnp.full((16,), b, jnp.int32)][0]

    def mT_step(i, mv):
        for g in range(G):
            mv = jnp.maximum(mv, _group(tr_v, i, g))
        return mv

    mT_vec = lax.fori_loop(0, T, mT_step, jnp.full((16,), -jnp.inf, jnp.float32))
    mT = jnp.max(mT_vec)

    def exp_step(i, carry):
        eg = [jnp.exp(_group(tr_v, i, g) - mT) for g in range(G)]
        b01 = plsc.pack(eg[0], eg[1], format=plsc.PackFormat.INTERLEAVED)
        b23 = plsc.pack(eg[2], eg[3], format=plsc.PackFormat.INTERLEAVED)
        trb_v[i, pl.ds(0, 16)] = plsc.bitcast(b01, jnp.int32)
        trb_v[i, pl.ds(16, 16)] = plsc.bitcast(b23, jnp.int32)
        return carry

    lax.fori_loop(0, T, exp_step, 0)
    em_cp.wait()

    c0 = [hd_v[pl.ds(g * 16, 16)] + _group(em_v, 0, g) for g in range(G)]
    m0 = jnp.max(_vmax4(c0))
    p_init = tuple(jnp.exp(c0[g] - m0) for g in range(G))

    def step(t, carry):
        k_acc, kv, p0, p1, p2, p3 = carry
        pc = (p0, p1, p2, p3)
        sp = [plsc.bitcast(
            plsc.pack(pc[g], pc[g], format=plsc.PackFormat.INTERLEAVED),
            jnp.int32) for g in range(G)]
        scale = plsc.bitcast((127 - kv) << 23, jnp.float32)
        e = [_group(em_v, t, g) for g in range(G)]
        ws = [jnp.exp(jnp.minimum(e[g], 80.0)) * scale for g in range(G)]
        q01 = jnp.zeros((32,), jnp.bfloat16)
        q23 = jnp.zeros((32,), jnp.bfloat16)
        for lane in range(16):
            idx = jnp.full((16,), lane, jnp.int32)
            for gs in range(G):
                pib = plsc.bitcast(sp[gs][idx], jnp.bfloat16)
                i = gs * 16 + lane
                e01 = plsc.bitcast(trb_v[i, pl.ds(0, 16)], jnp.bfloat16)
                e23 = plsc.bitcast(trb_v[i, pl.ds(16, 16)], jnp.bfloat16)
                q01 = q01 + pib * e01
                q23 = q23 + pib * e23
        q0, q1 = plsc.unpack(q01, format=plsc.PackFormat.INTERLEAVED)
        q2, q3 = plsc.unpack(q23, format=plsc.PackFormat.INTERLEAVED)
        q = [q0, q1, q2, q3]
        pnew = [q[g] * ws[g] for g in range(G)]
        pm = jnp.max(_vmax4(pnew))
        bits = plsc.bitcast(jnp.full((16,), pm), jnp.int32)
        kv_new = (bits >> 23) - 127
        return (k_acc + kv, kv_new, pnew[0], pnew[1], pnew[2], pnew[3])

    k_acc, _, f0, f1, f2, f3 = lax.fori_loop(
        1, seq_len, step,
        (jnp.zeros((16,), jnp.int32), jnp.zeros((16,), jnp.int32)) + p_init)
    pf = (f0, f1, f2, f3)
    s_acc = m0 + (seq_len - 1).astype(jnp.float32) * mT

    lg = [la_v[pl.ds(g * 16, 16)] for g in range(G)]
    mL = jnp.max(_vmax4(lg))
    z = [pf[g] * jnp.exp(lg[g] - mL) for g in range(G)]
    Z = jnp.sum(z[0] + z[1] + z[2] + z[3])

    zbits = plsc.bitcast(jnp.full((16,), Z), jnp.int32)
    ev = (zbits >> 23) - 127
    mant = plsc.bitcast((zbits & 0x007FFFFF) | 0x3F800000, jnp.float32)
    big = mant > SQRT2
    mant = jnp.where(big, mant * 0.5, mant)
    ev = jnp.where(big, ev + 1, ev)
    tt = (mant - 1.0) / (mant + 1.0)
    t2 = tt * tt
    lnm = tt * (2.0 + t2 * (2.0 / 3.0 + t2 * (2.0 / 5.0
                + t2 * (2.0 / 7.0 + t2 * (2.0 / 9.0)))))
    res = lnm + (ev + k_acc).astype(jnp.float32) * LN2 + (s_acc + mL)
    row_v[...] = res

    pltpu.sync_copy(row_v, shr_v.at[s])
    plsc.subcore_barrier()

    @pl.when(jnp.logical_and(c == 0, s == 0))
    def _():
        pltpu.sync_copy(shr_v, gat_v)
        col = plsc.load_gather(
            gat_v, [lax.iota(jnp.int32, 16), jnp.zeros((16,), jnp.int32)])
        row_v[...] = col
        pltpu.sync_copy(row_v, out_h)


def kernel(emissions, token_sizes, transitions, head_transitions,
           last_transitions):
    tok32 = token_sizes.astype(jnp.int32)
    mesh = plsc.VectorSubcoreMesh(core_axis_name="c", subcore_axis_name="s", num_cores=1)
    run = functools.partial(
        pl.kernel,
        out_type=jax.ShapeDtypeStruct((B,), jnp.float32),
        mesh=mesh,
        scratch_types=[
            pltpu.VMEM((S, T), jnp.float32),
            pltpu.VMEM((T, T), jnp.float32),
            pltpu.VMEM((T, 32), jnp.int32),
            pltpu.VMEM((B,), jnp.int32),
            pltpu.VMEM((T,), jnp.float32),
            pltpu.VMEM((T,), jnp.float32),
            pltpu.VMEM((16,), jnp.float32),
            pltpu.VMEM_SHARED((16, 16), jnp.float32),
            pltpu.VMEM((16, 16), jnp.float32),
            pltpu.SemaphoreType.DMA,
        ],
        compiler_params=pltpu.CompilerParams(needs_layout_passes=False),
    )(_crf_body)
    return run(emissions, tok32, transitions, head_transitions,
               last_transitions)
